# Initial kernel scaffold; baseline (speedup 1.0000x reference)
import jax
import jax.numpy as jnp
from jax.experimental import pallas as pl


def kernel(x, W, b):
    raise NotImplementedError("write your pallas kernel here")


# speedup vs baseline: 5.8589x; 5.8589x over previous
import functools

import jax
import jax.numpy as jnp
from jax.experimental import pallas as pl
from jax.experimental.pallas import tpu as pltpu

DIM = 768
NUM_EXPERTS = 64
CAPACITY_FACTOR = 1.25
EPSILON = 1e-06
AUX_LOSS_WEIGHT = 0.01

ROW_BLOCK = 1024


def _gate_kernel(x_ref, wt_ref, b_ref, out_ref, aux_ref,
                 denom_acc, imp_acc, load_acc, *, n_rows, n_blocks, capacity):
    i = pl.program_id(0)

    @pl.when(i == 0)
    def _init():
        denom_acc[...] = jnp.zeros_like(denom_acc)
        imp_acc[...] = jnp.zeros_like(imp_acc)
        load_acc[...] = jnp.zeros_like(load_acc)

    logits = jnp.dot(x_ref[...], wt_ref[...],
                     preferred_element_type=jnp.float32) + b_ref[...]
    m = jnp.max(logits, axis=-1, keepdims=True)
    e = jnp.exp(logits - m)
    raw = e / jnp.sum(e, axis=-1, keepdims=True)

    iota = jax.lax.broadcasted_iota(jnp.int32, raw.shape, 1)
    m1 = jnp.max(raw, axis=-1, keepdims=True)
    i1 = jnp.min(jnp.where(raw == m1, iota, NUM_EXPERTS), axis=-1, keepdims=True)
    mask1 = iota == i1
    raw2 = jnp.where(mask1, -1.0, raw)
    m2 = jnp.max(raw2, axis=-1, keepdims=True)
    i2 = jnp.min(jnp.where(raw2 == m2, iota, NUM_EXPERTS), axis=-1, keepdims=True)
    mask = (mask1 | (iota == i2)).astype(jnp.float32)

    masked = raw * mask
    denom_acc[...] += jnp.sum(masked, axis=0, keepdims=True)
    imp_acc[...] += jnp.sum(raw, axis=0, keepdims=True)
    load_acc[...] += jnp.sum(mask, axis=0, keepdims=True)

    out_ref[pl.ds(i * ROW_BLOCK, ROW_BLOCK), :] = masked

    @pl.when(i == n_blocks - 1)
    def _finish():
        scale = capacity / (denom_acc[...] + EPSILON)
        out_ref[...] = out_ref[...] * scale
        imp = imp_acc[...] / n_rows
        load = load_acc[...] / n_rows
        aux = (AUX_LOSS_WEIGHT * jnp.mean(imp * load, keepdims=True)
               * (NUM_EXPERTS ** 2))
        aux_ref[...] = aux.reshape(1, 1)


def kernel(x, W, b):
    n, d = x.shape
    n_blocks = n // ROW_BLOCK
    capacity = float(int(CAPACITY_FACTOR * n))
    wt = W.T
    b2 = b.reshape(1, NUM_EXPERTS)

    gate, aux = pl.pallas_call(
        functools.partial(_gate_kernel, n_rows=float(n), n_blocks=n_blocks,
                          capacity=capacity),
        grid=(n_blocks,),
        in_specs=[
            pl.BlockSpec((ROW_BLOCK, d), lambda i: (i, 0)),
            pl.BlockSpec((d, NUM_EXPERTS), lambda i: (0, 0)),
            pl.BlockSpec((1, NUM_EXPERTS), lambda i: (0, 0)),
        ],
        out_specs=[
            pl.BlockSpec((n, NUM_EXPERTS), lambda i: (0, 0)),
            pl.BlockSpec((1, 1), lambda i: (0, 0)),
        ],
        out_shape=[
            jax.ShapeDtypeStruct((n, NUM_EXPERTS), jnp.float32),
            jax.ShapeDtypeStruct((1, 1), jnp.float32),
        ],
        scratch_shapes=[
            pltpu.VMEM((1, NUM_EXPERTS), jnp.float32),
            pltpu.VMEM((1, NUM_EXPERTS), jnp.float32),
            pltpu.VMEM((1, NUM_EXPERTS), jnp.float32),
        ],
        compiler_params=pltpu.CompilerParams(
            dimension_semantics=("arbitrary",),
        ),
    )(x, wt, b2)
    return gate, aux[0, 0]


# speedup vs baseline: 6.0595x; 1.0342x over previous
import functools

import jax
import jax.numpy as jnp
from jax.experimental import pallas as pl
from jax.experimental.pallas import tpu as pltpu

DIM = 768
NUM_EXPERTS = 64
CAPACITY_FACTOR = 1.25
EPSILON = 1e-06
AUX_LOSS_WEIGHT = 0.01

ROW_BLOCK = 1024


def _gate_kernel(x_ref, wt_ref, b_ref, out_ref, aux_ref,
                 denom_acc, imp_acc, load_acc, *, n_rows, n_blocks, capacity):
    i = pl.program_id(0)

    @pl.when(i == 0)
    def _init():
        denom_acc[...] = jnp.zeros_like(denom_acc)
        imp_acc[...] = jnp.zeros_like(imp_acc)
        load_acc[...] = jnp.zeros_like(load_acc)

    logits = jnp.dot(x_ref[...], wt_ref[...],
                     preferred_element_type=jnp.float32) + b_ref[...]

    iota = jax.lax.broadcasted_iota(jnp.int32, logits.shape, 1)
    m1 = jnp.max(logits, axis=-1, keepdims=True)
    i1 = jnp.min(jnp.where(logits == m1, iota, NUM_EXPERTS),
                 axis=-1, keepdims=True)
    mask1 = iota == i1
    l2 = jnp.where(mask1, -jnp.inf, logits)
    m2 = jnp.max(l2, axis=-1, keepdims=True)
    i2 = jnp.min(jnp.where(l2 == m2, iota, NUM_EXPERTS),
                 axis=-1, keepdims=True)
    mask = (mask1 | (iota == i2)).astype(jnp.float32)

    e = jnp.exp(logits - m1)
    rs = 1.0 / jnp.sum(e, axis=-1, keepdims=True)
    raw = e * rs
    masked = raw * mask
    denom_acc[...] += jnp.sum(masked, axis=0, keepdims=True)
    imp_acc[...] += jnp.sum(raw, axis=0, keepdims=True)
    load_acc[...] += jnp.sum(mask, axis=0, keepdims=True)

    out_ref[pl.ds(i * ROW_BLOCK, ROW_BLOCK), :] = masked

    @pl.when(i == n_blocks - 1)
    def _finish():
        scale = capacity / (denom_acc[...] + EPSILON)
        out_ref[...] = out_ref[...] * scale
        imp = imp_acc[...] / n_rows
        load = load_acc[...] / n_rows
        aux = (AUX_LOSS_WEIGHT * jnp.mean(imp * load, keepdims=True)
               * (NUM_EXPERTS ** 2))
        aux_ref[...] = aux.reshape(1, 1)


def kernel(x, W, b):
    n, d = x.shape
    n_blocks = n // ROW_BLOCK
    capacity = float(int(CAPACITY_FACTOR * n))
    wt = W.T
    b2 = b.reshape(1, NUM_EXPERTS)

    gate, aux = pl.pallas_call(
        functools.partial(_gate_kernel, n_rows=float(n), n_blocks=n_blocks,
                          capacity=capacity),
        grid=(n_blocks,),
        in_specs=[
            pl.BlockSpec((ROW_BLOCK, d), lambda i: (i, 0)),
            pl.BlockSpec((d, NUM_EXPERTS), lambda i: (0, 0)),
            pl.BlockSpec((1, NUM_EXPERTS), lambda i: (0, 0)),
        ],
        out_specs=[
            pl.BlockSpec((n, NUM_EXPERTS), lambda i: (0, 0)),
            pl.BlockSpec((1, 1), lambda i: (0, 0)),
        ],
        out_shape=[
            jax.ShapeDtypeStruct((n, NUM_EXPERTS), jnp.float32),
            jax.ShapeDtypeStruct((1, 1), jnp.float32),
        ],
        scratch_shapes=[
            pltpu.VMEM((1, NUM_EXPERTS), jnp.float32),
            pltpu.VMEM((1, NUM_EXPERTS), jnp.float32),
            pltpu.VMEM((1, NUM_EXPERTS), jnp.float32),
        ],
        compiler_params=pltpu.CompilerParams(
            dimension_semantics=("arbitrary",),
        ),
    )(x, wt, b2)
    return gate, aux[0, 0]
